# baseline (speedup 1.0000x reference)
import functools

import jax
import jax.numpy as jnp
from jax import lax
from jax.experimental import pallas as pl
from jax.experimental.pallas import tpu as pltpu
from jax.experimental.pallas import tpu_sc as plsc

D = 64
B = 4096 * 200
NC = 2
NS = 16
NW = NC * NS
B_PER_W = B // NW
C = 128
N_CHUNKS = B_PER_W // C

_mesh = plsc.VectorSubcoreMesh(core_axis_name="c", subcore_axis_name="s")


@functools.partial(
    pl.kernel,
    mesh=_mesh,
    out_type=jax.ShapeDtypeStruct((B, D), jnp.float32),
    scratch_types=[
        pltpu.VMEM((C,), jnp.int32),
        pltpu.VMEM((C, D), jnp.float32),
        pltpu.SemaphoreType.DMA,
    ],
    compiler_params=pltpu.CompilerParams(use_tc_tiling_on_sc=False),
)
def _gather_kernel(tok_hbm, table_hbm, out_hbm, idx_v, rows_v, sem):
    wid = lax.axis_index("s") * NC + lax.axis_index("c")
    base = wid * B_PER_W

    def chunk(i, carry):
        off = base + i * C
        pltpu.sync_copy(tok_hbm.at[pl.ds(off, C)], idx_v)
        pltpu.async_copy(table_hbm.at[idx_v], rows_v, sem).wait()
        pltpu.sync_copy(rows_v, out_hbm.at[pl.ds(off, C)])
        return carry

    lax.fori_loop(0, N_CHUNKS, chunk, 0)


def kernel(tokens, table):
    tok = tokens.reshape(-1).astype(jnp.int32)
    out = _gather_kernel(tok, table)
    return out.reshape(tokens.shape[0], tokens.shape[1], D)


# speedup vs baseline: 1.1933x; 1.1933x over previous
import functools

import jax
import jax.numpy as jnp
from jax import lax
from jax.experimental import pallas as pl
from jax.experimental.pallas import tpu as pltpu
from jax.experimental.pallas import tpu_sc as plsc

D = 64
B = 4096 * 200
NC = 2
NS = 16
NW = NC * NS
B_PER_W = B // NW
C = 128
K = 5
KC = K * C
G = B_PER_W // KC
NBUF = 2

_mesh = plsc.VectorSubcoreMesh(core_axis_name="c", subcore_axis_name="s")


@functools.partial(
    pl.kernel,
    mesh=_mesh,
    out_type=jax.ShapeDtypeStruct((B, D), jnp.float32),
    scratch_types=[
        pltpu.VMEM((NBUF, KC), jnp.int32),
        pltpu.VMEM((NBUF, KC, D), jnp.float32),
        pltpu.SemaphoreType.DMA,
        pltpu.SemaphoreType.DMA,
        pltpu.SemaphoreType.DMA,
        pltpu.SemaphoreType.DMA,
        pltpu.SemaphoreType.DMA,
    ],
    compiler_params=pltpu.CompilerParams(use_tc_tiling_on_sc=False),
)
def _gather_kernel(tok_hbm, table_hbm, out_hbm, idx_v, rows_v,
                   sem_i0, sem_i1, sem_g, sem_s0, sem_s1):
    wid = lax.axis_index("s") * NC + lax.axis_index("c")
    base = wid * B_PER_W
    sem_i = (sem_i0, sem_i1)
    sem_s = (sem_s0, sem_s1)

    def fire_gathers(b):
        return [
            pltpu.async_copy(
                table_hbm.at[idx_v.at[b, pl.ds(j * C, C)]],
                rows_v.at[b, pl.ds(j * C, C)],
                sem_g,
            )
            for j in range(K)
        ]

    for b in range(NBUF):
        pltpu.async_copy(tok_hbm.at[pl.ds(base + b * KC, KC)],
                         idx_v.at[b], sem_i[b])

    for b in range(NBUF):
        pltpu.make_async_copy(tok_hbm.at[pl.ds(base + b * KC, KC)],
                              idx_v.at[b], sem_i[b]).wait()
        hs = fire_gathers(b)
        for h in hs:
            h.wait()
        pltpu.async_copy(tok_hbm.at[pl.ds(base + (b + NBUF) * KC, KC)],
                         idx_v.at[b], sem_i[b])
        pltpu.async_copy(rows_v.at[b],
                         out_hbm.at[pl.ds(base + b * KC, KC)], sem_s[b])

    def grp(s, carry):
        for b in range(NBUF):
            blk = s * NBUF + b
            off = base + blk * KC
            pltpu.make_async_copy(tok_hbm.at[pl.ds(off, KC)],
                                  idx_v.at[b], sem_i[b]).wait()
            pltpu.make_async_copy(rows_v.at[b],
                                  out_hbm.at[pl.ds(off, KC)], sem_s[b]).wait()
            hs = fire_gathers(b)
            for h in hs:
                h.wait()
            off_pf = lax.rem(off + NBUF * KC, B)
            pltpu.async_copy(tok_hbm.at[pl.ds(off_pf, KC)],
                             idx_v.at[b], sem_i[b])
            pltpu.async_copy(rows_v.at[b],
                             out_hbm.at[pl.ds(off, KC)], sem_s[b])
        return carry

    lax.fori_loop(1, G // NBUF, grp, 0)

    for b in range(NBUF):
        pltpu.make_async_copy(rows_v.at[b],
                              out_hbm.at[pl.ds(base, KC)], sem_s[b]).wait()
        pltpu.make_async_copy(tok_hbm.at[pl.ds(base, KC)],
                              idx_v.at[b], sem_i[b]).wait()


def kernel(tokens, table):
    tok = tokens.reshape(-1).astype(jnp.int32)
    out = _gather_kernel(tok, table)
    return out.reshape(tokens.shape[0], tokens.shape[1], D)
